# baseline (speedup 1.0000x reference)
import jax
import jax.numpy as jnp
from jax import lax
from jax.experimental import pallas as pl
from jax.experimental.pallas import tpu as pltpu
from jax.experimental.pallas import tpu_sc as plsc

_NC = 2
_NS = 16
_NW = _NC * _NS
_C = 32


def _sc_body(seq_len, batch, d_model, x_hbm, t_hbm, o_hbm,
             xbuf0, xbuf1, tbuf, lsem0, lsem1, ssem0, ssem1):
    wid = lax.axis_index("s") * _NC + lax.axis_index("c")
    rows_per_worker = seq_len // _NW
    nchunk = rows_per_worker // _C
    words = _C * d_model
    tbase = wid * rows_per_worker

    xbufs = (xbuf0, xbuf1)
    lsems = (lsem0, lsem1)
    ssems = (ssem0, ssem1)
    n_iters = nchunk * batch

    def x_off(it):
        j, b = divmod(it, batch)
        return (b * seq_len + tbase + j * _C) * d_model

    loads = [None, None]
    stores = [None, None]
    loads[0] = pltpu.async_copy(
        x_hbm.at[pl.ds(x_off(0), words)], xbufs[0], lsems[0])

    for j in range(nchunk):
        pltpu.sync_copy(
            t_hbm.at[pl.ds((tbase + j * _C) * d_model, words)], tbuf)
        for b in range(batch):
            it = j * batch + b
            cur = it % 2
            nxt = 1 - cur
            if it + 1 < n_iters:
                if stores[nxt] is not None:
                    stores[nxt].wait()
                    stores[nxt] = None
                loads[nxt] = pltpu.async_copy(
                    x_hbm.at[pl.ds(x_off(it + 1), words)], xbufs[nxt],
                    lsems[nxt])
            loads[cur].wait()
            loads[cur] = None

            buf = xbufs[cur]

            @plsc.parallel_loop(0, words, 16, unroll=8)
            def _(i):
                s = pl.ds(i, 16)
                buf[s] = buf[s] + tbuf[s]

            stores[cur] = pltpu.async_copy(
                buf, o_hbm.at[pl.ds(x_off(it), words)], ssems[cur])

    for k in range(2):
        if stores[k] is not None:
            stores[k].wait()


def kernel(x, pos_table):
    batch, seq_len, d_model = x.shape
    words = _C * d_model
    xf = x.reshape(-1)
    tf = pos_table.reshape(-1)

    import functools
    body = functools.partial(_sc_body, seq_len, batch, d_model)
    out = pl.kernel(
        body,
        out_type=jax.ShapeDtypeStruct((batch * seq_len * d_model,), x.dtype),
        mesh=plsc.VectorSubcoreMesh(core_axis_name="c", subcore_axis_name="s"),
        scratch_types=[
            pltpu.VMEM((words,), jnp.float32),
            pltpu.VMEM((words,), jnp.float32),
            pltpu.VMEM((words,), jnp.float32),
            pltpu.SemaphoreType.DMA,
            pltpu.SemaphoreType.DMA,
            pltpu.SemaphoreType.DMA,
            pltpu.SemaphoreType.DMA,
        ],
    )(xf, tf)
    return out.reshape(batch, seq_len, d_model)


# speedup vs baseline: 2.7091x; 2.7091x over previous
import functools

import jax
import jax.numpy as jnp
from jax import lax
from jax.experimental import pallas as pl
from jax.experimental.pallas import tpu as pltpu
from jax.experimental.pallas import tpu_sc as plsc

_NC = 2
_NS = 16
_NW = _NC * _NS
_C = 32


def _sc_body(seq_len, batch, d_model, x_hbm, t_hbm, o_hbm,
             xbuf0, xbuf1, tbuf, lsem0, lsem1, ssem0, ssem1):
    wid = lax.axis_index("s") * _NC + lax.axis_index("c")
    rows_per_worker = seq_len // _NW
    nchunk = rows_per_worker // _C
    tbase = wid * rows_per_worker

    xbufs = (xbuf0, xbuf1)
    lsems = (lsem0, lsem1)
    ssems = (ssem0, ssem1)
    n_iters = nchunk * batch

    def x_row(it):
        j, b = divmod(it, batch)
        return b * seq_len + tbase + j * _C

    loads = [None, None]
    stores = [None, None]
    loads[0] = pltpu.async_copy(
        x_hbm.at[pl.ds(x_row(0), _C), :], xbufs[0], lsems[0])

    for j in range(nchunk):
        pltpu.sync_copy(t_hbm.at[pl.ds(tbase + j * _C, _C), :], tbuf)
        for b in range(batch):
            it = j * batch + b
            cur = it % 2
            nxt = 1 - cur
            if it + 1 < n_iters:
                if stores[nxt] is not None:
                    stores[nxt].wait()
                    stores[nxt] = None
                loads[nxt] = pltpu.async_copy(
                    x_hbm.at[pl.ds(x_row(it + 1), _C), :], xbufs[nxt],
                    lsems[nxt])
            loads[cur].wait()
            loads[cur] = None

            buf = xbufs[cur]

            slices_per_row = d_model // 16

            @plsc.parallel_loop(0, _C * slices_per_row, 1, unroll=8)
            def _(i):
                r = i // slices_per_row
                k = i - r * slices_per_row
                s = pl.ds(k * 16, 16)
                buf[r, s] = buf[r, s] + tbuf[r, s]

            stores[cur] = pltpu.async_copy(
                buf, o_hbm.at[pl.ds(x_row(it), _C), :], ssems[cur])

    for k in range(2):
        if stores[k] is not None:
            stores[k].wait()


def kernel(x, pos_table):
    batch, seq_len, d_model = x.shape
    xf = x.reshape(batch * seq_len, d_model)

    body = functools.partial(_sc_body, seq_len, batch, d_model)
    out = pl.kernel(
        body,
        out_type=jax.ShapeDtypeStruct((batch * seq_len, d_model), x.dtype),
        mesh=plsc.VectorSubcoreMesh(core_axis_name="c", subcore_axis_name="s"),
        scratch_types=[
            pltpu.VMEM((_C, d_model), jnp.float32),
            pltpu.VMEM((_C, d_model), jnp.float32),
            pltpu.VMEM((_C, d_model), jnp.float32),
            pltpu.SemaphoreType.DMA,
            pltpu.SemaphoreType.DMA,
            pltpu.SemaphoreType.DMA,
            pltpu.SemaphoreType.DMA,
        ],
    )(xf, pos_table)
    return out.reshape(batch, seq_len, d_model)


# speedup vs baseline: 3.1396x; 1.1589x over previous
import functools

import jax
import jax.numpy as jnp
from jax import lax
from jax.experimental import pallas as pl
from jax.experimental.pallas import tpu as pltpu
from jax.experimental.pallas import tpu_sc as plsc

_NC = 2
_NS = 16
_NW = _NC * _NS
_C = 16
_R = 4


def _sc_body(seq_len, batch, d_model, x_hbm, t_hbm, o_hbm,
             xbufs, tbufs, lsems, tsems, ssems):
    wid = lax.axis_index("s") * _NC + lax.axis_index("c")
    rows_per_worker = seq_len // _NW
    nchunk = rows_per_worker // _C
    tbase = wid * rows_per_worker
    n_iters = nchunk * batch
    slices_per_row = d_model // 16

    def x_row(it):
        j, b = divmod(it, batch)
        return b * seq_len + tbase + j * _C

    def t_load(j, slot):
        return pltpu.async_copy(
            t_hbm.at[pl.ds(tbase + j * _C, _C), :], tbufs[slot], tsems[slot])

    loads = [None] * _R
    stores = [None] * _R
    tloads = [None, None]

    tloads[0] = t_load(0, 0)
    for p in range(min(_R - 1, n_iters)):
        loads[p] = pltpu.async_copy(
            x_hbm.at[pl.ds(x_row(p), _C), :], xbufs[p], lsems[p])

    for it in range(n_iters):
        j, b = divmod(it, batch)
        sl = it % _R
        ts = j % 2
        if b == 0:
            tloads[ts].wait()
            tloads[ts] = None
            if j + 1 < nchunk:
                tloads[1 - ts] = t_load(j + 1, 1 - ts)
        loads[sl].wait()
        loads[sl] = None

        buf = xbufs[sl]
        tbuf = tbufs[ts]

        @plsc.parallel_loop(0, _C * slices_per_row, 1, unroll=8)
        def _(i):
            r = i // slices_per_row
            k = i - r * slices_per_row
            s = pl.ds(k * 16, 16)
            buf[r, s] = buf[r, s] + tbuf[r, s]

        stores[sl] = pltpu.async_copy(
            buf, o_hbm.at[pl.ds(x_row(it), _C), :], ssems[sl])

        nit = it + _R - 1
        if nit < n_iters:
            nsl = nit % _R
            if stores[nsl] is not None:
                stores[nsl].wait()
                stores[nsl] = None
            loads[nsl] = pltpu.async_copy(
                x_hbm.at[pl.ds(x_row(nit), _C), :], xbufs[nsl], lsems[nsl])

    for s in stores:
        if s is not None:
            s.wait()


def kernel(x, pos_table):
    batch, seq_len, d_model = x.shape
    xf = x.reshape(batch * seq_len, d_model)

    def body(x_hbm, t_hbm, o_hbm, *scratch):
        xbufs = scratch[0:_R]
        tbufs = scratch[_R:_R + 2]
        lsems = scratch[_R + 2:2 * _R + 2]
        tsems = scratch[2 * _R + 2:2 * _R + 4]
        ssems = scratch[2 * _R + 4:3 * _R + 4]
        _sc_body(seq_len, batch, d_model, x_hbm, t_hbm, o_hbm,
                 xbufs, tbufs, lsems, tsems, ssems)

    out = pl.kernel(
        body,
        out_type=jax.ShapeDtypeStruct((batch * seq_len, d_model), x.dtype),
        mesh=plsc.VectorSubcoreMesh(core_axis_name="c", subcore_axis_name="s"),
        scratch_types=(
            [pltpu.VMEM((_C, d_model), jnp.float32)] * _R
            + [pltpu.VMEM((_C, d_model), jnp.float32)] * 2
            + [pltpu.SemaphoreType.DMA] * _R
            + [pltpu.SemaphoreType.DMA] * 2
            + [pltpu.SemaphoreType.DMA] * _R
        ),
    )(xf, pos_table)
    return out.reshape(batch, seq_len, d_model)


# speedup vs baseline: 3.2074x; 1.0216x over previous
import jax
import jax.numpy as jnp
from jax import lax
from jax.experimental import pallas as pl
from jax.experimental.pallas import tpu as pltpu
from jax.experimental.pallas import tpu_sc as plsc

_NC = 2
_NS = 16
_NW = _NC * _NS
_C = 8
_G = 3


def _sc_body(seq_len, batch, d_model, x_hbm, t_hbm, o_hbm,
             xbufs, tbufs, lsems, tsems, ssems):
    wid = lax.axis_index("s") * _NC + lax.axis_index("c")
    rows_per_worker = seq_len // _NW
    nchunk = rows_per_worker // _C
    tbase = wid * rows_per_worker
    slices_per_row = d_model // 16

    def row0(j, b):
        return b * seq_len + tbase + j * _C

    def issue_loads(j, s):
        for b in range(batch):
            lsl = s * batch + b
            if stores[lsl] is not None:
                stores[lsl].wait()
                stores[lsl] = None
            loads[lsl] = pltpu.async_copy(
                x_hbm.at[pl.ds(row0(j, b), _C), :], xbufs[lsl], lsems[lsl])
        tloads[s] = pltpu.async_copy(
            t_hbm.at[pl.ds(tbase + j * _C, _C), :], tbufs[s], tsems[s])

    loads = [None] * (_G * batch)
    stores = [None] * (_G * batch)
    tloads = [None] * _G

    issue_loads(0, 0)
    if nchunk > 1:
        issue_loads(1, 1)

    for j in range(nchunk):
        s = j % _G
        if j + 2 < nchunk:
            issue_loads(j + 2, (j + 2) % _G)
        tloads[s].wait()
        tloads[s] = None
        for b in range(batch):
            loads[s * batch + b].wait()
            loads[s * batch + b] = None

        tbuf = tbufs[s]
        bufs = tuple(xbufs[s * batch + b] for b in range(batch))

        @plsc.parallel_loop(0, _C * slices_per_row, 1, unroll=4)
        def _(i):
            r = i // slices_per_row
            k = i - r * slices_per_row
            sl = pl.ds(k * 16, 16)
            t16 = tbuf[r, sl]
            for b in range(batch):
                bufs[b][r, sl] = bufs[b][r, sl] + t16

        for b in range(batch):
            lsl = s * batch + b
            stores[lsl] = pltpu.async_copy(
                xbufs[lsl], o_hbm.at[pl.ds(row0(j, b), _C), :], ssems[lsl])

    for st in stores:
        if st is not None:
            st.wait()


def kernel(x, pos_table):
    batch, seq_len, d_model = x.shape
    xf = x.reshape(batch * seq_len, d_model)
    nbuf = _G * batch

    def body(x_hbm, t_hbm, o_hbm, *scratch):
        xbufs = scratch[0:nbuf]
        tbufs = scratch[nbuf:nbuf + _G]
        lsems = scratch[nbuf + _G:2 * nbuf + _G]
        tsems = scratch[2 * nbuf + _G:2 * nbuf + 2 * _G]
        ssems = scratch[2 * nbuf + 2 * _G:3 * nbuf + 2 * _G]
        _sc_body(seq_len, batch, d_model, x_hbm, t_hbm, o_hbm,
                 xbufs, tbufs, lsems, tsems, ssems)

    out = pl.kernel(
        body,
        out_type=jax.ShapeDtypeStruct((batch * seq_len, d_model), x.dtype),
        mesh=plsc.VectorSubcoreMesh(core_axis_name="c", subcore_axis_name="s"),
        scratch_types=(
            [pltpu.VMEM((_C, d_model), jnp.float32)] * nbuf
            + [pltpu.VMEM((_C, d_model), jnp.float32)] * _G
            + [pltpu.SemaphoreType.DMA] * nbuf
            + [pltpu.SemaphoreType.DMA] * _G
            + [pltpu.SemaphoreType.DMA] * nbuf
        ),
    )(xf, pos_table)
    return out.reshape(batch, seq_len, d_model)


# speedup vs baseline: 4.3910x; 1.3690x over previous
import jax
import jax.numpy as jnp
from jax.experimental import pallas as pl
from jax.experimental.pallas import tpu as pltpu

_SBLK = 512


def _add_kernel(x_ref, pos_ref, o_ref):
    o_ref[...] = x_ref[...] + pos_ref[...][None, :, :]


def kernel(x, pos_table):
    batch, seq_len, d_model = x.shape
    grid = (seq_len // _SBLK,)
    return pl.pallas_call(
        _add_kernel,
        grid=grid,
        in_specs=[
            pl.BlockSpec((batch, _SBLK, d_model), lambda i: (0, i, 0)),
            pl.BlockSpec((_SBLK, d_model), lambda i: (i, 0)),
        ],
        out_specs=pl.BlockSpec((batch, _SBLK, d_model), lambda i: (0, i, 0)),
        out_shape=jax.ShapeDtypeStruct((batch, seq_len, d_model), x.dtype),
        compiler_params=pltpu.CompilerParams(
            dimension_semantics=("parallel",),
        ),
    )(x, pos_table)
